# Initial kernel scaffold; baseline (speedup 1.0000x reference)
import jax
import jax.numpy as jnp
from jax.experimental import pallas as pl


def kernel(audio_codes, tables, W1, b1, W2, b2, W3, b3, W4, b4):
    raise NotImplementedError("write your pallas kernel here")


# speedup vs baseline: 9.9729x; 9.9729x over previous
import functools

import jax
import jax.numpy as jnp
from jax import lax
from jax.experimental import pallas as pl
from jax.experimental.pallas import tpu as pltpu
from jax.experimental.pallas import tpu_sc as plsc

NUM_CODEBOOKS = 9
CODEBOOK_SIZE = 1088
EMB_DIM = 64
SEQ_LEN = 131072

NW = 32
LOOKUPS = NUM_CODEBOOKS * SEQ_LEN
ROWS_PER_WORKER = LOOKUPS // NW
SUB = 128
CHUNK = 1024
SUBS_PER_CHUNK = CHUNK // SUB
CHUNKS = ROWS_PER_WORKER // CHUNK


def _sc_gather(codes2d, tables_flat):
    mesh = plsc.VectorSubcoreMesh(core_axis_name="c", subcore_axis_name="s")

    @functools.partial(
        pl.kernel,
        mesh=mesh,
        out_type=jax.ShapeDtypeStruct((LOOKUPS, EMB_DIM), jnp.float32),
        scratch_types=[
            pltpu.VMEM((SUBS_PER_CHUNK, SUB), jnp.int32),
            pltpu.VMEM((CHUNK, EMB_DIM), jnp.float32),
            pltpu.SemaphoreType.DMA,
        ],
        compiler_params=pltpu.CompilerParams(use_tc_tiling_on_sc=False),
    )
    def k(codes_ref, tables_ref, out_ref, idx_v, rows_v, sem):
        wid = lax.axis_index("c") * 16 + lax.axis_index("s")
        worker_row0 = wid * (ROWS_PER_WORKER // SUB)

        def chunk_body(c, carry):
            row_base = worker_row0 + c * SUBS_PER_CHUNK
            pltpu.sync_copy(codes_ref.at[pl.ds(row_base, SUBS_PER_CHUNK)], idx_v)
            copies = [
                pltpu.async_copy(
                    tables_ref.at[idx_v.at[j]],
                    rows_v.at[pl.ds(j * SUB, SUB)],
                    sem,
                )
                for j in range(SUBS_PER_CHUNK)
            ]
            for cp in copies:
                cp.wait()
            pltpu.sync_copy(rows_v, out_ref.at[pl.ds(row_base * SUB, CHUNK)])
            return carry

        lax.fori_loop(0, CHUNKS, chunk_body, 0)

    return k(codes2d, tables_flat)


BLK = 1024
H1, H2, H3 = 512, 256, 128


def _mlp_body(e_ref, w1_ref, b1_ref, w2_ref, b2_ref, w3_ref, b3_ref,
              w4_ref, b4_ref, o_ref):
    acc = jnp.zeros((BLK, H1), jnp.float32) + b1_ref[...]
    for i in range(NUM_CODEBOOKS):
        acc = acc + jnp.dot(e_ref[i], w1_ref[i],
                            preferred_element_type=jnp.float32)
    h1 = jnp.maximum(acc, 0.0)
    h2 = jnp.maximum(
        jnp.dot(h1, w2_ref[...], preferred_element_type=jnp.float32)
        + b2_ref[...], 0.0)
    h3 = jnp.maximum(
        jnp.dot(h2, w3_ref[...], preferred_element_type=jnp.float32)
        + b3_ref[...], 0.0)
    o_ref[...] = jnp.tanh(
        jnp.dot(h3, w4_ref[...], preferred_element_type=jnp.float32)
        + b4_ref[...])


def _tc_mlp(embs, w1, b1, w2, b2, w3, b3, w4, b4, interpret=False):
    grid = (SEQ_LEN // BLK,)
    return pl.pallas_call(
        _mlp_body,
        grid=grid,
        in_specs=[
            pl.BlockSpec((NUM_CODEBOOKS, BLK, EMB_DIM), lambda j: (0, j, 0)),
            pl.BlockSpec((NUM_CODEBOOKS, EMB_DIM, H1), lambda j: (0, 0, 0)),
            pl.BlockSpec((1, H1), lambda j: (0, 0)),
            pl.BlockSpec((H1, H2), lambda j: (0, 0)),
            pl.BlockSpec((1, H2), lambda j: (0, 0)),
            pl.BlockSpec((H2, H3), lambda j: (0, 0)),
            pl.BlockSpec((1, H3), lambda j: (0, 0)),
            pl.BlockSpec((H3, 1), lambda j: (0, 0)),
            pl.BlockSpec((1, 1), lambda j: (0, 0)),
        ],
        out_specs=pl.BlockSpec((BLK, 1), lambda j: (j, 0)),
        out_shape=jax.ShapeDtypeStruct((SEQ_LEN, 1), jnp.float32),
        interpret=interpret,
    )(embs, w1, b1, w2, b2, w3, b3, w4, b4)


def kernel(audio_codes, tables, W1, b1, W2, b2, W3, b3, W4, b4):
    codes = audio_codes.astype(jnp.int32)
    offs = (jnp.arange(NUM_CODEBOOKS, dtype=jnp.int32) * CODEBOOK_SIZE)[:, None]
    codes2d = (codes + offs).reshape(LOOKUPS // SUB, SUB)
    tables_flat = tables.reshape(NUM_CODEBOOKS * CODEBOOK_SIZE, EMB_DIM)
    embs = _sc_gather(codes2d, tables_flat).reshape(
        NUM_CODEBOOKS, SEQ_LEN, EMB_DIM)
    out = _tc_mlp(
        embs,
        W1.reshape(NUM_CODEBOOKS, EMB_DIM, H1), b1.reshape(1, H1),
        W2, b2.reshape(1, H2),
        W3, b3.reshape(1, H3),
        W4, b4.reshape(1, 1),
    )
    return out.reshape(SEQ_LEN)


# speedup vs baseline: 14.9830x; 1.5024x over previous
import functools

import jax
import jax.numpy as jnp
from jax import lax
from jax.experimental import pallas as pl
from jax.experimental.pallas import tpu as pltpu
from jax.experimental.pallas import tpu_sc as plsc

NUM_CODEBOOKS = 9
CODEBOOK_SIZE = 1088
EMB_DIM = 64
SEQ_LEN = 131072

NW = 32
LOOKUPS = NUM_CODEBOOKS * SEQ_LEN
ROWS_PER_WORKER = LOOKUPS // NW
SUB = 128
CHUNK = 1024
SUBS_PER_CHUNK = CHUNK // SUB
CHUNKS = ROWS_PER_WORKER // CHUNK


def _sc_gather(codes2d, tables_flat):
    mesh = plsc.VectorSubcoreMesh(core_axis_name="c", subcore_axis_name="s")

    @functools.partial(
        pl.kernel,
        mesh=mesh,
        out_type=jax.ShapeDtypeStruct((LOOKUPS, EMB_DIM), jnp.float32),
        scratch_types=[
            pltpu.VMEM((SUBS_PER_CHUNK, SUB), jnp.int32),
            pltpu.VMEM((CHUNK, EMB_DIM), jnp.float32),
            pltpu.SemaphoreType.DMA,
        ],
        compiler_params=pltpu.CompilerParams(use_tc_tiling_on_sc=False),
    )
    def k(codes_ref, tables_ref, out_ref, idx_v, rows_v, sem):
        wid = lax.axis_index("c") * 16 + lax.axis_index("s")
        worker_row0 = wid * (ROWS_PER_WORKER // SUB)

        def chunk_body(c, carry):
            row_base = worker_row0 + c * SUBS_PER_CHUNK
            pltpu.sync_copy(codes_ref.at[pl.ds(row_base, SUBS_PER_CHUNK)], idx_v)
            copies = [
                pltpu.async_copy(
                    tables_ref.at[idx_v.at[j]],
                    rows_v.at[pl.ds(j * SUB, SUB)],
                    sem,
                )
                for j in range(SUBS_PER_CHUNK)
            ]
            for cp in copies:
                cp.wait()
            pltpu.sync_copy(rows_v, out_ref.at[pl.ds(row_base * SUB, CHUNK)])
            return carry

        lax.fori_loop(0, CHUNKS, chunk_body, 0)

    return k(codes2d, tables_flat)


BLK = 1024
HB = BLK // 2
H1, H2, H3 = 512, 256, 128


def _mlp_body(e_ref, w1_ref, b1_ref, w2_ref, b2_ref, w3_ref, b3_ref,
              w4_ref, b4_ref, o_ref):
    acc_e = jnp.zeros((HB, H1), jnp.float32) + b1_ref[...]
    acc_o = jnp.zeros((HB, H1), jnp.float32) + b1_ref[...]
    for i in range(NUM_CODEBOOKS):
        pair = e_ref[i]
        acc_e = acc_e + jnp.dot(pair[:, :EMB_DIM], w1_ref[i],
                                preferred_element_type=jnp.float32)
        acc_o = acc_o + jnp.dot(pair[:, EMB_DIM:], w1_ref[i],
                                preferred_element_type=jnp.float32)
    for half, acc in ((0, acc_e), (1, acc_o)):
        h = jnp.maximum(acc, 0.0)
        h = jnp.maximum(
            jnp.dot(h, w2_ref[...], preferred_element_type=jnp.float32)
            + b2_ref[...], 0.0)
        h = jnp.maximum(
            jnp.dot(h, w3_ref[...], preferred_element_type=jnp.float32)
            + b3_ref[...], 0.0)
        y = jnp.tanh(
            jnp.dot(h, w4_ref[...], preferred_element_type=jnp.float32)
            + b4_ref[...])
        o_ref[:, half] = y[:, 0]


def _tc_mlp(embs, w1, b1, w2, b2, w3, b3, w4, b4, interpret=False):
    grid = (SEQ_LEN // BLK,)
    return pl.pallas_call(
        _mlp_body,
        grid=grid,
        in_specs=[
            pl.BlockSpec((NUM_CODEBOOKS, HB, 2 * EMB_DIM), lambda j: (0, j, 0)),
            pl.BlockSpec((NUM_CODEBOOKS, EMB_DIM, H1), lambda j: (0, 0, 0)),
            pl.BlockSpec((1, H1), lambda j: (0, 0)),
            pl.BlockSpec((H1, H2), lambda j: (0, 0)),
            pl.BlockSpec((1, H2), lambda j: (0, 0)),
            pl.BlockSpec((H2, H3), lambda j: (0, 0)),
            pl.BlockSpec((1, H3), lambda j: (0, 0)),
            pl.BlockSpec((H3, 1), lambda j: (0, 0)),
            pl.BlockSpec((1, 1), lambda j: (0, 0)),
        ],
        out_specs=pl.BlockSpec((HB, 2), lambda j: (j, 0)),
        out_shape=jax.ShapeDtypeStruct((SEQ_LEN // 2, 2), jnp.float32),
        interpret=interpret,
    )(embs, w1, b1, w2, b2, w3, b3, w4, b4)


def kernel(audio_codes, tables, W1, b1, W2, b2, W3, b3, W4, b4):
    codes = audio_codes.astype(jnp.int32)
    offs = (jnp.arange(NUM_CODEBOOKS, dtype=jnp.int32) * CODEBOOK_SIZE)[:, None]
    codes2d = (codes + offs).reshape(LOOKUPS // SUB, SUB)
    tables_flat = tables.reshape(NUM_CODEBOOKS * CODEBOOK_SIZE, EMB_DIM)
    embs = _sc_gather(codes2d, tables_flat).reshape(
        NUM_CODEBOOKS, SEQ_LEN // 2, 2 * EMB_DIM)
    out = _tc_mlp(
        embs,
        W1.reshape(NUM_CODEBOOKS, EMB_DIM, H1), b1.reshape(1, H1),
        W2, b2.reshape(1, H2),
        W3, b3.reshape(1, H3),
        W4, b4.reshape(1, 1),
    )
    return out.reshape(SEQ_LEN)


# speedup vs baseline: 14.9865x; 1.0002x over previous
import functools

import jax
import jax.numpy as jnp
from jax import lax
from jax.experimental import pallas as pl
from jax.experimental.pallas import tpu as pltpu
from jax.experimental.pallas import tpu_sc as plsc

NUM_CODEBOOKS = 9
CODEBOOK_SIZE = 1088
EMB_DIM = 64
SEQ_LEN = 131072

NW = 32
LOOKUPS = NUM_CODEBOOKS * SEQ_LEN
ROWS_PER_WORKER = LOOKUPS // NW
SUB = 128
CHUNK = 1024
SUBS_PER_CHUNK = CHUNK // SUB
CHUNKS = ROWS_PER_WORKER // CHUNK


def _sc_gather(codes2d, tables_flat):
    mesh = plsc.VectorSubcoreMesh(core_axis_name="c", subcore_axis_name="s")

    @functools.partial(
        pl.kernel,
        mesh=mesh,
        out_type=jax.ShapeDtypeStruct((LOOKUPS, EMB_DIM), jnp.float32),
        scratch_types=[
            pltpu.VMEM((SUBS_PER_CHUNK, SUB), jnp.int32),
            pltpu.VMEM((CHUNK, EMB_DIM), jnp.float32),
            pltpu.SemaphoreType.DMA,
        ],
        compiler_params=pltpu.CompilerParams(use_tc_tiling_on_sc=False),
    )
    def k(codes_ref, tables_ref, out_ref, idx_v, rows_v, sem):
        wid = lax.axis_index("c") * 16 + lax.axis_index("s")
        worker_row0 = wid * (ROWS_PER_WORKER // SUB)

        def chunk_body(c, carry):
            row_base = worker_row0 + c * SUBS_PER_CHUNK
            pltpu.sync_copy(codes_ref.at[pl.ds(row_base, SUBS_PER_CHUNK)], idx_v)
            copies = [
                pltpu.async_copy(
                    tables_ref.at[idx_v.at[j]],
                    rows_v.at[pl.ds(j * SUB, SUB)],
                    sem,
                )
                for j in range(SUBS_PER_CHUNK)
            ]
            for cp in copies:
                cp.wait()
            pltpu.sync_copy(rows_v, out_ref.at[pl.ds(row_base * SUB, CHUNK)])
            return carry

        lax.fori_loop(0, CHUNKS, chunk_body, 0)

    return k(codes2d, tables_flat)


BLK = 1024
HB = BLK // 2
H1, H2, H3 = 512, 256, 128


def _mlp_body(e_ref, w1_ref, b1_ref, w2_ref, b2_ref, w3_ref, b3_ref,
              w4_ref, b4_ref, o_ref):
    acc_e = jnp.zeros((HB, H1), jnp.float32) + b1_ref[...]
    acc_o = jnp.zeros((HB, H1), jnp.float32) + b1_ref[...]
    for i in range(NUM_CODEBOOKS):
        pair = e_ref[i].astype(jnp.bfloat16)
        acc_e = acc_e + jnp.dot(pair[:, :EMB_DIM], w1_ref[i],
                                preferred_element_type=jnp.float32)
        acc_o = acc_o + jnp.dot(pair[:, EMB_DIM:], w1_ref[i],
                                preferred_element_type=jnp.float32)
    for half, acc in ((0, acc_e), (1, acc_o)):
        h = jnp.maximum(acc, 0.0)
        h = jnp.maximum(
            jnp.dot(h, w2_ref[...], preferred_element_type=jnp.float32)
            + b2_ref[...], 0.0)
        h = jnp.maximum(
            jnp.dot(h, w3_ref[...], preferred_element_type=jnp.float32)
            + b3_ref[...], 0.0)
        y = jnp.tanh(
            jnp.dot(h, w4_ref[...], preferred_element_type=jnp.float32)
            + b4_ref[...])
        o_ref[:, half] = y[:, 0]


def _tc_mlp(embs, w1, b1, w2, b2, w3, b3, w4, b4, interpret=False):
    grid = (SEQ_LEN // BLK,)
    return pl.pallas_call(
        _mlp_body,
        grid=grid,
        in_specs=[
            pl.BlockSpec((NUM_CODEBOOKS, HB, 2 * EMB_DIM), lambda j: (0, j, 0)),
            pl.BlockSpec((NUM_CODEBOOKS, EMB_DIM, H1), lambda j: (0, 0, 0)),
            pl.BlockSpec((1, H1), lambda j: (0, 0)),
            pl.BlockSpec((H1, H2), lambda j: (0, 0)),
            pl.BlockSpec((1, H2), lambda j: (0, 0)),
            pl.BlockSpec((H2, H3), lambda j: (0, 0)),
            pl.BlockSpec((1, H3), lambda j: (0, 0)),
            pl.BlockSpec((H3, 1), lambda j: (0, 0)),
            pl.BlockSpec((1, 1), lambda j: (0, 0)),
        ],
        out_specs=pl.BlockSpec((HB, 2), lambda j: (j, 0)),
        out_shape=jax.ShapeDtypeStruct((SEQ_LEN // 2, 2), jnp.float32),
        interpret=interpret,
    )(embs, w1, b1, w2, b2, w3, b3, w4, b4)


def kernel(audio_codes, tables, W1, b1, W2, b2, W3, b3, W4, b4):
    codes = audio_codes.astype(jnp.int32)
    offs = (jnp.arange(NUM_CODEBOOKS, dtype=jnp.int32) * CODEBOOK_SIZE)[:, None]
    codes2d = (codes + offs).reshape(LOOKUPS // SUB, SUB)
    tables_flat = tables.reshape(NUM_CODEBOOKS * CODEBOOK_SIZE, EMB_DIM)
    embs = _sc_gather(codes2d, tables_flat).reshape(
        NUM_CODEBOOKS, SEQ_LEN // 2, 2 * EMB_DIM)
    out = _tc_mlp(
        embs,
        W1.astype(jnp.bfloat16).reshape(NUM_CODEBOOKS, EMB_DIM, H1),
        b1.reshape(1, H1),
        W2, b2.reshape(1, H2),
        W3, b3.reshape(1, H3),
        W4, b4.reshape(1, 1),
    )
    return out.reshape(SEQ_LEN)


# speedup vs baseline: 15.7807x; 1.0530x over previous
import functools

import jax
import jax.numpy as jnp
from jax import lax
from jax.experimental import pallas as pl
from jax.experimental.pallas import tpu as pltpu
from jax.experimental.pallas import tpu_sc as plsc

NUM_CODEBOOKS = 9
CODEBOOK_SIZE = 1088
EMB_DIM = 64
SEQ_LEN = 131072

NW = 32
LOOKUPS = NUM_CODEBOOKS * SEQ_LEN
ROWS_PER_WORKER = LOOKUPS // NW
SUB = 128
CHUNK = 1024
SUBS_PER_CHUNK = CHUNK // SUB
CHUNKS = ROWS_PER_WORKER // CHUNK


def _sc_gather(codes2d, tables_flat):
    mesh = plsc.VectorSubcoreMesh(core_axis_name="c", subcore_axis_name="s")

    @functools.partial(
        pl.kernel,
        mesh=mesh,
        out_type=jax.ShapeDtypeStruct((LOOKUPS, EMB_DIM), jnp.float32),
        scratch_types=[
            pltpu.VMEM((SUBS_PER_CHUNK, SUB), jnp.int32),
            pltpu.VMEM((CHUNK, EMB_DIM), jnp.float32),
            pltpu.SemaphoreType.DMA,
        ],
        compiler_params=pltpu.CompilerParams(use_tc_tiling_on_sc=False),
    )
    def k(codes_ref, tables_ref, out_ref, idx_v, rows_v, sem):
        wid = lax.axis_index("c") * 16 + lax.axis_index("s")
        worker_row0 = wid * (ROWS_PER_WORKER // SUB)

        def chunk_body(c, carry):
            row_base = worker_row0 + c * SUBS_PER_CHUNK
            pltpu.sync_copy(codes_ref.at[pl.ds(row_base, SUBS_PER_CHUNK)], idx_v)
            copies = [
                pltpu.async_copy(
                    tables_ref.at[idx_v.at[j]],
                    rows_v.at[pl.ds(j * SUB, SUB)],
                    sem,
                )
                for j in range(SUBS_PER_CHUNK)
            ]
            for cp in copies:
                cp.wait()
            pltpu.sync_copy(rows_v, out_ref.at[pl.ds(row_base * SUB, CHUNK)])
            return carry

        lax.fori_loop(0, CHUNKS, chunk_body, 0)

    return k(codes2d, tables_flat)


BLK = 2048
HB = BLK // 2
H1, H2, H3 = 512, 256, 128


def _mlp_body(e_ref, w1_ref, b1_ref, w2_ref, b2_ref, w3_ref, b3_ref,
              w4_ref, b4_ref, o_ref):
    acc_e = jnp.zeros((HB, H1), jnp.float32) + b1_ref[...]
    acc_o = jnp.zeros((HB, H1), jnp.float32) + b1_ref[...]
    for i in range(NUM_CODEBOOKS):
        pair = e_ref[i].astype(jnp.bfloat16)
        acc_e = acc_e + jnp.dot(pair[:, :EMB_DIM], w1_ref[i],
                                preferred_element_type=jnp.float32)
        acc_o = acc_o + jnp.dot(pair[:, EMB_DIM:], w1_ref[i],
                                preferred_element_type=jnp.float32)
    for half, acc in ((0, acc_e), (1, acc_o)):
        h = jnp.maximum(acc, 0.0)
        h = jnp.maximum(
            jnp.dot(h, w2_ref[...], preferred_element_type=jnp.float32)
            + b2_ref[...], 0.0)
        h = jnp.maximum(
            jnp.dot(h, w3_ref[...], preferred_element_type=jnp.float32)
            + b3_ref[...], 0.0)
        y = jnp.tanh(
            jnp.dot(h, w4_ref[...], preferred_element_type=jnp.float32)
            + b4_ref[...])
        o_ref[:, half] = y[:, 0]


def _tc_mlp(embs, w1, b1, w2, b2, w3, b3, w4, b4, interpret=False):
    grid = (SEQ_LEN // BLK,)
    return pl.pallas_call(
        _mlp_body,
        grid=grid,
        in_specs=[
            pl.BlockSpec((NUM_CODEBOOKS, HB, 2 * EMB_DIM), lambda j: (0, j, 0)),
            pl.BlockSpec((NUM_CODEBOOKS, EMB_DIM, H1), lambda j: (0, 0, 0)),
            pl.BlockSpec((1, H1), lambda j: (0, 0)),
            pl.BlockSpec((H1, H2), lambda j: (0, 0)),
            pl.BlockSpec((1, H2), lambda j: (0, 0)),
            pl.BlockSpec((H2, H3), lambda j: (0, 0)),
            pl.BlockSpec((1, H3), lambda j: (0, 0)),
            pl.BlockSpec((H3, 1), lambda j: (0, 0)),
            pl.BlockSpec((1, 1), lambda j: (0, 0)),
        ],
        out_specs=pl.BlockSpec((HB, 2), lambda j: (j, 0)),
        out_shape=jax.ShapeDtypeStruct((SEQ_LEN // 2, 2), jnp.float32),
        interpret=interpret,
    )(embs, w1, b1, w2, b2, w3, b3, w4, b4)


def kernel(audio_codes, tables, W1, b1, W2, b2, W3, b3, W4, b4):
    codes = audio_codes.astype(jnp.int32)
    offs = (jnp.arange(NUM_CODEBOOKS, dtype=jnp.int32) * CODEBOOK_SIZE)[:, None]
    codes2d = (codes + offs).reshape(LOOKUPS // SUB, SUB)
    tables_flat = tables.reshape(NUM_CODEBOOKS * CODEBOOK_SIZE, EMB_DIM)
    embs = _sc_gather(codes2d, tables_flat).reshape(
        NUM_CODEBOOKS, SEQ_LEN // 2, 2 * EMB_DIM)
    out = _tc_mlp(
        embs,
        W1.astype(jnp.bfloat16).reshape(NUM_CODEBOOKS, EMB_DIM, H1),
        b1.reshape(1, H1),
        W2, b2.reshape(1, H2),
        W3, b3.reshape(1, H3),
        W4, b4.reshape(1, 1),
    )
    return out.reshape(SEQ_LEN)


# speedup vs baseline: 16.1388x; 1.0227x over previous
import functools

import jax
import jax.numpy as jnp
from jax import lax
from jax.experimental import pallas as pl
from jax.experimental.pallas import tpu as pltpu
from jax.experimental.pallas import tpu_sc as plsc

NUM_CODEBOOKS = 9
CODEBOOK_SIZE = 1088
EMB_DIM = 64
SEQ_LEN = 131072

NW = 32
LOOKUPS = NUM_CODEBOOKS * SEQ_LEN
ROWS_PER_WORKER = LOOKUPS // NW
SUB = 128
CHUNK = 1024
SUBS_PER_CHUNK = CHUNK // SUB
CHUNKS = ROWS_PER_WORKER // CHUNK


def _sc_gather(codes2d, tables_flat):
    mesh = plsc.VectorSubcoreMesh(core_axis_name="c", subcore_axis_name="s")

    @functools.partial(
        pl.kernel,
        mesh=mesh,
        out_type=jax.ShapeDtypeStruct((LOOKUPS, EMB_DIM), jnp.float32),
        scratch_types=[
            pltpu.VMEM((SUBS_PER_CHUNK, SUB), jnp.int32),
            pltpu.VMEM((CHUNK, EMB_DIM), jnp.float32),
            pltpu.SemaphoreType.DMA,
        ],
        compiler_params=pltpu.CompilerParams(use_tc_tiling_on_sc=False),
    )
    def k(codes_ref, tables_ref, out_ref, idx_v, rows_v, sem):
        wid = lax.axis_index("c") * 16 + lax.axis_index("s")
        worker_row0 = wid * (ROWS_PER_WORKER // SUB)

        def chunk_body(c, carry):
            row_base = worker_row0 + c * SUBS_PER_CHUNK
            pltpu.sync_copy(codes_ref.at[pl.ds(row_base, SUBS_PER_CHUNK)], idx_v)
            copies = [
                pltpu.async_copy(
                    tables_ref.at[idx_v.at[j]],
                    rows_v.at[pl.ds(j * SUB, SUB)],
                    sem,
                )
                for j in range(SUBS_PER_CHUNK)
            ]
            for cp in copies:
                cp.wait()
            pltpu.sync_copy(rows_v, out_ref.at[pl.ds(row_base * SUB, CHUNK)])
            return carry

        lax.fori_loop(0, CHUNKS, chunk_body, 0)

    return k(codes2d, tables_flat)


BLK = 4096
HB = BLK // 2
H1, H2, H3 = 512, 256, 128


def _mlp_body(e_ref, w1_ref, b1_ref, w2_ref, b2_ref, w3_ref, b3_ref,
              w4_ref, b4_ref, o_ref):
    acc_e = jnp.zeros((HB, H1), jnp.float32) + b1_ref[...]
    acc_o = jnp.zeros((HB, H1), jnp.float32) + b1_ref[...]
    for i in range(NUM_CODEBOOKS):
        pair = e_ref[i].astype(jnp.bfloat16)
        acc_e = acc_e + jnp.dot(pair[:, :EMB_DIM], w1_ref[i],
                                preferred_element_type=jnp.float32)
        acc_o = acc_o + jnp.dot(pair[:, EMB_DIM:], w1_ref[i],
                                preferred_element_type=jnp.float32)
    for half, acc in ((0, acc_e), (1, acc_o)):
        h = jnp.maximum(acc, 0.0)
        h = jnp.maximum(
            jnp.dot(h, w2_ref[...], preferred_element_type=jnp.float32)
            + b2_ref[...], 0.0)
        h = jnp.maximum(
            jnp.dot(h, w3_ref[...], preferred_element_type=jnp.float32)
            + b3_ref[...], 0.0)
        y = jnp.tanh(
            jnp.dot(h, w4_ref[...], preferred_element_type=jnp.float32)
            + b4_ref[...])
        o_ref[:, half] = y[:, 0]


def _tc_mlp(embs, w1, b1, w2, b2, w3, b3, w4, b4, interpret=False):
    grid = (SEQ_LEN // BLK,)
    return pl.pallas_call(
        _mlp_body,
        grid=grid,
        in_specs=[
            pl.BlockSpec((NUM_CODEBOOKS, HB, 2 * EMB_DIM), lambda j: (0, j, 0)),
            pl.BlockSpec((NUM_CODEBOOKS, EMB_DIM, H1), lambda j: (0, 0, 0)),
            pl.BlockSpec((1, H1), lambda j: (0, 0)),
            pl.BlockSpec((H1, H2), lambda j: (0, 0)),
            pl.BlockSpec((1, H2), lambda j: (0, 0)),
            pl.BlockSpec((H2, H3), lambda j: (0, 0)),
            pl.BlockSpec((1, H3), lambda j: (0, 0)),
            pl.BlockSpec((H3, 1), lambda j: (0, 0)),
            pl.BlockSpec((1, 1), lambda j: (0, 0)),
        ],
        out_specs=pl.BlockSpec((HB, 2), lambda j: (j, 0)),
        out_shape=jax.ShapeDtypeStruct((SEQ_LEN // 2, 2), jnp.float32),
        interpret=interpret,
    )(embs, w1, b1, w2, b2, w3, b3, w4, b4)


def kernel(audio_codes, tables, W1, b1, W2, b2, W3, b3, W4, b4):
    codes = audio_codes.astype(jnp.int32)
    offs = (jnp.arange(NUM_CODEBOOKS, dtype=jnp.int32) * CODEBOOK_SIZE)[:, None]
    codes2d = (codes + offs).reshape(LOOKUPS // SUB, SUB)
    tables_flat = tables.reshape(NUM_CODEBOOKS * CODEBOOK_SIZE, EMB_DIM)
    embs = _sc_gather(codes2d, tables_flat).reshape(
        NUM_CODEBOOKS, SEQ_LEN // 2, 2 * EMB_DIM)
    out = _tc_mlp(
        embs,
        W1.astype(jnp.bfloat16).reshape(NUM_CODEBOOKS, EMB_DIM, H1),
        b1.reshape(1, H1),
        W2, b2.reshape(1, H2),
        W3, b3.reshape(1, H3),
        W4, b4.reshape(1, 1),
    )
    return out.reshape(SEQ_LEN)


# speedup vs baseline: 18.6442x; 1.1552x over previous
import functools

import jax
import jax.numpy as jnp
from jax import lax
from jax.experimental import pallas as pl
from jax.experimental.pallas import tpu as pltpu
from jax.experimental.pallas import tpu_sc as plsc

NUM_CODEBOOKS = 9
CODEBOOK_SIZE = 1088
EMB_DIM = 64
SEQ_LEN = 131072

NW = 32
LOOKUPS = NUM_CODEBOOKS * SEQ_LEN
SUB = 128
SUBS_PER_CHUNK = 8
STEP = SUBS_PER_CHUNK * SUB

NCH = 4
TCH = SEQ_LEN // NCH
CODE_ROWS_PER_CB = SEQ_LEN // SUB
CHUNK_CODE_ROWS = TCH // SUB


def _sc_gather_chunk(codes2d, tables_flat, chunk):
    mesh = plsc.VectorSubcoreMesh(core_axis_name="c", subcore_axis_name="s")

    @functools.partial(
        pl.kernel,
        mesh=mesh,
        out_type=jax.ShapeDtypeStruct((NUM_CODEBOOKS * TCH, EMB_DIM),
                                      jnp.float32),
        scratch_types=[
            pltpu.VMEM((SUBS_PER_CHUNK, SUB), jnp.int32),
            pltpu.VMEM((STEP, EMB_DIM), jnp.float32),
            pltpu.SemaphoreType.DMA,
        ],
        compiler_params=pltpu.CompilerParams(use_tc_tiling_on_sc=False),
    )
    def k(codes_ref, tables_ref, out_ref, idx_v, rows_v, sem):
        wid = lax.axis_index("c") * 16 + lax.axis_index("s")

        def seg_body(i, carry):
            code_row = (i * CODE_ROWS_PER_CB + chunk * CHUNK_CODE_ROWS
                        + wid * SUBS_PER_CHUNK)
            out_row = i * TCH + wid * STEP
            pltpu.sync_copy(codes_ref.at[pl.ds(code_row, SUBS_PER_CHUNK)],
                            idx_v)
            copies = [
                pltpu.async_copy(
                    tables_ref.at[idx_v.at[j]],
                    rows_v.at[pl.ds(j * SUB, SUB)],
                    sem,
                )
                for j in range(SUBS_PER_CHUNK)
            ]
            for cp in copies:
                cp.wait()
            pltpu.sync_copy(rows_v, out_ref.at[pl.ds(out_row, STEP)])
            return carry

        lax.fori_loop(0, NUM_CODEBOOKS, seg_body, 0)

    return k(codes2d, tables_flat)


BLK = 4096
HB = BLK // 2
H1, H2, H3 = 512, 256, 128


def _mlp_body(e_ref, w1_ref, b1_ref, w2_ref, b2_ref, w3_ref, b3_ref,
              w4_ref, b4_ref, o_ref):
    acc_e = jnp.zeros((HB, H1), jnp.float32) + b1_ref[...]
    acc_o = jnp.zeros((HB, H1), jnp.float32) + b1_ref[...]
    for i in range(NUM_CODEBOOKS):
        pair = e_ref[i].astype(jnp.bfloat16)
        acc_e = acc_e + jnp.dot(pair[:, :EMB_DIM], w1_ref[i],
                                preferred_element_type=jnp.float32)
        acc_o = acc_o + jnp.dot(pair[:, EMB_DIM:], w1_ref[i],
                                preferred_element_type=jnp.float32)
    for half, acc in ((0, acc_e), (1, acc_o)):
        h = jnp.maximum(acc, 0.0)
        h = jnp.maximum(
            jnp.dot(h, w2_ref[...], preferred_element_type=jnp.float32)
            + b2_ref[...], 0.0)
        h = jnp.maximum(
            jnp.dot(h, w3_ref[...], preferred_element_type=jnp.float32)
            + b3_ref[...], 0.0)
        y = jnp.tanh(
            jnp.dot(h, w4_ref[...], preferred_element_type=jnp.float32)
            + b4_ref[...])
        o_ref[:, half] = y[:, 0]


def _tc_mlp(embs, w1, b1, w2, b2, w3, b3, w4, b4, interpret=False):
    grid = (TCH // BLK,)
    return pl.pallas_call(
        _mlp_body,
        grid=grid,
        in_specs=[
            pl.BlockSpec((NUM_CODEBOOKS, HB, 2 * EMB_DIM), lambda j: (0, j, 0)),
            pl.BlockSpec((NUM_CODEBOOKS, EMB_DIM, H1), lambda j: (0, 0, 0)),
            pl.BlockSpec((1, H1), lambda j: (0, 0)),
            pl.BlockSpec((H1, H2), lambda j: (0, 0)),
            pl.BlockSpec((1, H2), lambda j: (0, 0)),
            pl.BlockSpec((H2, H3), lambda j: (0, 0)),
            pl.BlockSpec((1, H3), lambda j: (0, 0)),
            pl.BlockSpec((H3, 1), lambda j: (0, 0)),
            pl.BlockSpec((1, 1), lambda j: (0, 0)),
        ],
        out_specs=pl.BlockSpec((HB, 2), lambda j: (j, 0)),
        out_shape=jax.ShapeDtypeStruct((TCH // 2, 2), jnp.float32),
        interpret=interpret,
    )(embs, w1, b1, w2, b2, w3, b3, w4, b4)


def kernel(audio_codes, tables, W1, b1, W2, b2, W3, b3, W4, b4):
    codes = audio_codes.astype(jnp.int32)
    offs = (jnp.arange(NUM_CODEBOOKS, dtype=jnp.int32) * CODEBOOK_SIZE)[:, None]
    codes2d = (codes + offs).reshape(LOOKUPS // SUB, SUB)
    tables_flat = tables.reshape(NUM_CODEBOOKS * CODEBOOK_SIZE, EMB_DIM)
    w1 = W1.astype(jnp.bfloat16).reshape(NUM_CODEBOOKS, EMB_DIM, H1)
    b1r, b2r, b3r, b4r = (b1.reshape(1, H1), b2.reshape(1, H2),
                          b3.reshape(1, H3), b4.reshape(1, 1))
    outs = []
    for chunk in range(NCH):
        embs = _sc_gather_chunk(codes2d, tables_flat, chunk).reshape(
            NUM_CODEBOOKS, TCH // 2, 2 * EMB_DIM)
        outs.append(_tc_mlp(embs, w1, b1r, W2, b2r, W3, b3r, W4, b4r))
    return jnp.concatenate(outs, axis=0).reshape(SEQ_LEN)
